# Initial kernel scaffold; baseline (speedup 1.0000x reference)
import jax
import jax.numpy as jnp
from jax.experimental import pallas as pl


def kernel(x):
    raise NotImplementedError("write your pallas kernel here")


# speedup vs baseline: 1.5897x; 1.5897x over previous
import functools

import jax
import jax.numpy as jnp
import numpy as np
from jax.experimental import pallas as pl
from jax.experimental.pallas import tpu as pltpu

_ANCHORS = np.array([[116.0, 90.0], [156.0, 198.0], [373.0, 326.0]], dtype=np.float32)
_NUM_CLASSES = 80
_IMG_DIM = 608.0


def _decode_kernel(x_ref, o_ref, *, G, stride, anchors):
    a = pl.program_id(1)
    X = x_ref[0, 0]
    GG = G * G
    nch = _NUM_CLASSES + 5

    sig = jax.nn.sigmoid(X)

    top = X[0:8]
    row8 = jax.lax.broadcasted_iota(jnp.int32, (8, GG), 0)
    col = jax.lax.broadcasted_iota(jnp.int32, (1, GG), 1)
    gy = (col // G).astype(jnp.float32)
    gx = (col % G).astype(jnp.float32)

    ex = jnp.exp(top)
    sig8 = sig[0:8]

    aw = jnp.where(a == 0, anchors[0, 0], jnp.where(a == 1, anchors[1, 0], anchors[2, 0]))
    ah = jnp.where(a == 0, anchors[0, 1], jnp.where(a == 1, anchors[1, 1], anchors[2, 1]))

    base = jnp.where((row8 == 2) | (row8 == 3), ex, sig8)
    add = jnp.where(row8 == 0, gx, jnp.where(row8 == 1, gy, 0.0))
    scale = jnp.where(
        row8 < 2, stride, jnp.where(row8 == 2, aw, jnp.where(row8 == 3, ah, 1.0))
    )
    top_out = (base + add) * scale

    y = jnp.concatenate([top_out, sig[8:]], axis=0)
    o_ref[0] = y.T


def kernel(x):
    B = x.shape[0]
    G = x.shape[2]
    nA = _ANCHORS.shape[0]
    nch = _NUM_CLASSES + 5
    GG = G * G
    stride = _IMG_DIM / G

    xr = x.reshape(B, nA, nch, GG)

    out = pl.pallas_call(
        functools.partial(_decode_kernel, G=G, stride=stride, anchors=_ANCHORS),
        grid=(B, nA),
        in_specs=[pl.BlockSpec((1, 1, nch, GG), lambda b, a: (b, a, 0, 0))],
        out_specs=pl.BlockSpec((1, GG, nch), lambda b, a: (b * nA + a, 0, 0)),
        out_shape=jax.ShapeDtypeStruct((B * nA, GG, nch), jnp.float32),
        compiler_params=pltpu.CompilerParams(
            dimension_semantics=("parallel", "arbitrary"),
        ),
    )(xr)

    return out.reshape(B, nA * GG, nch)
